# baseline (device time: 274455 ns/iter reference)
import jax
import jax.numpy as jnp
from jax import lax
from jax.experimental import pallas as pl
from jax.experimental.pallas import tpu as pltpu

N_DEV = 8
HALF = 256


def kernel(x, w_mat):
    m_per, k = x.shape
    _, n_per = w_mat.shape

    def body(
        x_ref, w_ref, out_ref,
        bufR3, bufL3, cbOwn, cbMid, cbFar, bufL2,
        bufR1, bufR2, bufL1,
        stage, outstage,
        send_sems, recv_sems, stage_sem, out_sems,
    ):
        my = lax.axis_index("i")

        def dev(i):
            i = i % N_DEV
            return jnp.where(i < 4, i, 11 - i)

        pos = dev(my)
        even = (pos % 2) == 0
        right = dev(pos + 1)
        leftd = dev(pos - 1)
        chordd = dev(pos + jnp.where(even, 3, -3))

        barrier_sem = pltpu.get_barrier_semaphore()
        for nbr in (leftd, right, chordd):
            pl.semaphore_signal(
                barrier_sem, inc=1,
                device_id=(nbr,), device_id_type=pl.DeviceIdType.MESH,
            )
        pl.semaphore_wait(barrier_sem, 3)

        def rdma(idx, src, dst, dev_id):
            return pltpu.make_async_remote_copy(
                src_ref=src, dst_ref=dst,
                send_sem=send_sems.at[idx], recv_sem=recv_sems.at[idx],
                device_id=(dev_id,), device_id_type=pl.DeviceIdType.MESH,
            )

        r1 = rdma(0, x_ref, bufR1, right)
        l1 = rdma(1, x_ref, bufL1, leftd)
        c0 = rdma(2, x_ref.at[pl.ds(HALF, HALF), :], cbOwn, chordd)
        r1.start()
        l1.start()
        c0.start()

        slot_ctr = [0]
        last_copy = [None, None]

        def gemm256(src_rows, origin, row_off):
            s = slot_ctr[0] % 2
            slot_ctr[0] += 1
            if last_copy[s] is not None:
                last_copy[s].wait()
            outstage[s, :, :] = jnp.dot(
                src_rows, w_ref[...], preferred_element_type=jnp.float32
            )
            cp = pltpu.make_async_copy(
                outstage.at[s],
                out_ref.at[pl.ds(origin * m_per + row_off, HALF), :],
                out_sems.at[s],
            )
            cp.start()
            last_copy[s] = cp

        def gemm_hbm(hbm_src, origin, row_off):
            cp = pltpu.make_async_copy(hbm_src, stage, stage_sem)
            cp.start()
            cp.wait()
            gemm256(stage[...], origin, row_off)

        gemm256(x_ref[0:HALF, :], my, 0)
        gemm256(x_ref[HALF:, :], my, HALF)

        c0.wait_recv()
        gemm_hbm(cbOwn, chordd, HALF)

        cmid_e = rdma(4, bufR1, cbMid, chordd)
        cmid_o = rdma(4, bufL1, cbMid, chordd)

        r1.wait_recv()
        r2 = rdma(3, bufR1, bufR2, right)
        r2.start()

        @pl.when(even)
        def _():
            cmid_e.start()

        l1.wait_recv()
        l2 = rdma(5, bufL1, bufL2, leftd)
        l2.start()

        @pl.when(jnp.logical_not(even))
        def _():
            cmid_o.start()

        gemm_hbm(bufR1.at[pl.ds(0, HALF), :], dev(pos - 1), 0)
        gemm_hbm(bufR1.at[pl.ds(HALF, HALF), :], dev(pos - 1), HALF)
        gemm_hbm(bufL1.at[pl.ds(0, HALF), :], dev(pos + 1), 0)
        gemm_hbm(bufL1.at[pl.ds(HALF, HALF), :], dev(pos + 1), HALF)

        cfar_e = rdma(7, bufR2.at[pl.ds(HALF, HALF), :], cbFar, chordd)
        cfar_o = rdma(7, bufL2.at[pl.ds(HALF, HALF), :], cbFar, chordd)

        r2.wait_recv()
        r3 = rdma(6, bufR2.at[pl.ds(0, HALF), :], bufR3, right)
        r3.start()

        @pl.when(even)
        def _():
            cfar_e.start()

        l2.wait_recv()
        l3 = rdma(8, bufL2.at[pl.ds(0, HALF), :], bufL3, leftd)
        l3.start()

        @pl.when(jnp.logical_not(even))
        def _():
            cfar_o.start()

        gemm_hbm(bufR2.at[pl.ds(0, HALF), :], dev(pos - 2), 0)
        gemm_hbm(bufR2.at[pl.ds(HALF, HALF), :], dev(pos - 2), HALF)
        gemm_hbm(bufL2.at[pl.ds(0, HALF), :], dev(pos + 2), 0)
        gemm_hbm(bufL2.at[pl.ds(HALF, HALF), :], dev(pos + 2), HALF)

        cmid_e.wait_recv()
        gemm_hbm(cbMid.at[pl.ds(0, HALF), :], dev(pos + 4), 0)
        gemm_hbm(cbMid.at[pl.ds(HALF, HALF), :], dev(pos + 4), HALF)

        r3.wait_recv()
        gemm_hbm(bufR3, dev(pos - 3), 0)
        l3.wait_recv()
        gemm_hbm(bufL3, dev(pos + 3), 0)
        cfar_e.wait_recv()
        far_origin = jnp.where(chordd == dev(pos + 3), dev(pos - 3), dev(pos + 3))
        gemm_hbm(cbFar, far_origin, HALF)

        for cp in last_copy:
            if cp is not None:
                cp.wait()
        for d in (r1, l1, c0, r2, l2, r3, l3):
            d.wait_send()
        cmid_e.wait_send()
        cfar_e.wait_send()

    halfbuf = jax.ShapeDtypeStruct((HALF, k), jnp.float32)
    any_spec = pl.BlockSpec(memory_space=pltpu.MemorySpace.HBM)
    outs = pl.pallas_call(
        body,
        out_shape=(
            jax.ShapeDtypeStruct((N_DEV * m_per, n_per), jnp.float32),
            halfbuf, halfbuf,
            halfbuf,
            jax.ShapeDtypeStruct((m_per, k), jnp.float32),
            halfbuf,
            jax.ShapeDtypeStruct((m_per, k), jnp.float32),
            jax.ShapeDtypeStruct((m_per, k), jnp.float32),
            jax.ShapeDtypeStruct((m_per, k), jnp.float32),
            jax.ShapeDtypeStruct((m_per, k), jnp.float32),
        ),
        in_specs=[
            pl.BlockSpec(memory_space=pltpu.VMEM),
            pl.BlockSpec(memory_space=pltpu.VMEM),
        ],
        out_specs=(
            any_spec,
            any_spec, any_spec, any_spec, any_spec, any_spec, any_spec,
            any_spec, any_spec, any_spec,
        ),
        scratch_shapes=[
            pltpu.VMEM((HALF, k), jnp.float32),
            pltpu.VMEM((2, HALF, n_per), jnp.float32),
            pltpu.SemaphoreType.DMA((9,)),
            pltpu.SemaphoreType.DMA((9,)),
            pltpu.SemaphoreType.DMA,
            pltpu.SemaphoreType.DMA((2,)),
        ],
        compiler_params=pltpu.CompilerParams(
            collective_id=0,
            vmem_limit_bytes=64 * 1024 * 1024,
        ),
    )(x, w_mat)
    return outs[0]


# device time: 272375 ns/iter; 1.0076x vs baseline; 1.0076x over previous
import jax
import jax.numpy as jnp
from jax import lax
from jax.experimental import pallas as pl
from jax.experimental.pallas import tpu as pltpu

N_DEV = 8
A_ROWS = 256


def kernel(x, w_mat):
    m_per, k = x.shape
    _, n_per = w_mat.shape
    bc = m_per - A_ROWS

    def body(
        x_ref, w_ref, out_ref,
        bufR1, bufR2, bufR3,
        bufL1, bufL2, bufL3,
        cbOwn, cbMid, cbFar,
        stage,
        send_sems, recv_sems, copy_sems,
    ):
        my = lax.axis_index("i")

        def dev(i):
            i = i % N_DEV
            return jnp.where(i < 4, i, 11 - i)

        pos = dev(my)
        even = (pos % 2) == 0
        right = dev(pos + 1)
        leftd = dev(pos - 1)
        chordd = dev(pos + jnp.where(even, 3, -3))

        barrier_sem = pltpu.get_barrier_semaphore()
        for nbr in (leftd, right, chordd):
            pl.semaphore_signal(
                barrier_sem, inc=1,
                device_id=(nbr,), device_id_type=pl.DeviceIdType.MESH,
            )
        pl.semaphore_wait(barrier_sem, 3)

        def rdma(idx, src, dst, dev_id):
            return pltpu.make_async_remote_copy(
                src_ref=src, dst_ref=dst,
                send_sem=send_sems.at[idx], recv_sem=recv_sems.at[idx],
                device_id=(dev_id,), device_id_type=pl.DeviceIdType.MESH,
            )

        r1 = rdma(0, x_ref, bufR1, right)
        l1 = rdma(1, x_ref, bufL1, leftd)
        c0 = rdma(2, x_ref.at[pl.ds(A_ROWS, bc), :], cbOwn, chordd)
        r1.start()
        l1.start()
        c0.start()

        out_ref[pl.ds(my * m_per, m_per), :] = jnp.dot(
            x_ref[...], w_ref[...], preferred_element_type=jnp.float32
        )

        pending = []

        def stage_piece(i, hbm_src, origin, row_off, nrows):
            slot = i % 2
            cp = pltpu.make_async_copy(
                hbm_src, stage.at[slot, pl.ds(0, nrows), :], copy_sems.at[slot]
            )
            cp.start()
            if pending:
                drain_one()
            pending.append((cp, slot, origin, row_off, nrows))

        def drain_one():
            cp, slot, origin, row_off, nrows = pending.pop(0)
            cp.wait()
            out_ref[pl.ds(origin * m_per + row_off, nrows), :] = jnp.dot(
                stage[slot, 0:nrows, :], w_ref[...],
                preferred_element_type=jnp.float32,
            )

        c0.wait_recv()
        stage_piece(0, cbOwn, chordd, A_ROWS, bc)
        drain_one()

        cmid_e = rdma(4, bufR1, cbMid, chordd)
        cmid_o = rdma(4, bufL1, cbMid, chordd)

        r1.wait_recv()
        r2 = rdma(3, bufR1, bufR2, right)
        r2.start()

        @pl.when(even)
        def _():
            cmid_e.start()

        l1.wait_recv()
        l2 = rdma(5, bufL1, bufL2, leftd)
        l2.start()

        @pl.when(jnp.logical_not(even))
        def _():
            cmid_o.start()

        stage_piece(1, bufR1, dev(pos - 1), 0, m_per)
        stage_piece(2, bufL1, dev(pos + 1), 0, m_per)
        drain_one()

        cfar_e = rdma(7, bufR2.at[pl.ds(A_ROWS, bc), :], cbFar, chordd)
        cfar_o = rdma(7, bufL2.at[pl.ds(A_ROWS, bc), :], cbFar, chordd)

        r2.wait_recv()
        r3 = rdma(6, bufR2.at[pl.ds(0, A_ROWS), :], bufR3, right)
        r3.start()

        @pl.when(even)
        def _():
            cfar_e.start()

        l2.wait_recv()
        l3 = rdma(8, bufL2.at[pl.ds(0, A_ROWS), :], bufL3, leftd)
        l3.start()

        @pl.when(jnp.logical_not(even))
        def _():
            cfar_o.start()

        stage_piece(3, bufR2, dev(pos - 2), 0, m_per)
        stage_piece(4, bufL2, dev(pos + 2), 0, m_per)
        cmid_e.wait_recv()
        stage_piece(5, cbMid, dev(pos + 4), 0, m_per)
        drain_one()

        r3.wait_recv()
        stage_piece(6, bufR3, dev(pos - 3), 0, A_ROWS)
        l3.wait_recv()
        stage_piece(7, bufL3, dev(pos + 3), 0, A_ROWS)
        cfar_e.wait_recv()
        far_origin = jnp.where(chordd == dev(pos + 3), dev(pos - 3), dev(pos + 3))
        stage_piece(8, cbFar, far_origin, A_ROWS, bc)

        while pending:
            drain_one()

        for d in (r1, l1, c0, r2, l2, r3, l3):
            d.wait_send()
        cmid_e.wait_send()
        cfar_e.wait_send()

    full = jax.ShapeDtypeStruct((m_per, k), jnp.float32)
    part = jax.ShapeDtypeStruct((bc, k), jnp.float32)
    any_spec = pl.BlockSpec(memory_space=pltpu.MemorySpace.HBM)
    outs = pl.pallas_call(
        body,
        out_shape=(
            jax.ShapeDtypeStruct((N_DEV * m_per, n_per), jnp.float32),
            full, full, jax.ShapeDtypeStruct((A_ROWS, k), jnp.float32),
            full, full, jax.ShapeDtypeStruct((A_ROWS, k), jnp.float32),
            part, full, part,
        ),
        in_specs=[
            pl.BlockSpec(memory_space=pltpu.VMEM),
            pl.BlockSpec(memory_space=pltpu.VMEM),
        ],
        out_specs=(
            pl.BlockSpec(memory_space=pltpu.VMEM),
            any_spec, any_spec, any_spec,
            any_spec, any_spec, any_spec,
            any_spec, any_spec, any_spec,
        ),
        scratch_shapes=[
            pltpu.VMEM((2, m_per, k), jnp.float32),
            pltpu.SemaphoreType.DMA((9,)),
            pltpu.SemaphoreType.DMA((9,)),
            pltpu.SemaphoreType.DMA((2,)),
        ],
        compiler_params=pltpu.CompilerParams(
            collective_id=0,
            vmem_limit_bytes=64 * 1024 * 1024,
        ),
    )(x, w_mat)
    return outs[0]
